# baseline (speedup 1.0000x reference)
import jax
import jax.numpy as jnp
from jax import lax
from jax.experimental import pallas as pl
from jax.experimental.pallas import tpu as pltpu
from jax.experimental.pallas import tpu_sc as plsc

NC = 2
NS = 16
NW = NC * NS
B = 80
L = 16


def _padded(n):
  q = NS * 128
  return (n + q - 1) // q * q


def _make_deg_kernel(n, e):
  k = e // NW // B
  np_ = _padded(n)
  stripe = np_ // NS
  W = 8
  mesh = plsc.VectorSubcoreMesh(core_axis_name="c", subcore_axis_name="s")

  def body(dst_hbm, out_hbm, dst_v, ones_v, zbuf, acc_sh, sem):
    c = lax.axis_index("c")
    s = lax.axis_index("s")
    wid = s * NC + c

    z16 = jnp.zeros((L,), jnp.float32)
    def zfill(i, carry):
      zbuf[pl.ds(i * L, L)] = z16
      return carry
    lax.fori_loop(0, stripe // L, zfill, 0)
    for i in range(B // L):
      ones_v[pl.ds(i * L, L)] = jnp.ones((L,), jnp.float32)
    pltpu.sync_copy(zbuf, acc_sh.at[pl.ds(s * stripe, stripe)])
    pltpu.sync_copy(dst_hbm.at[wid], dst_v)
    plsc.subcore_barrier()

    def chunk(j, carry):
      pltpu.async_copy(ones_v, acc_sh.at[dst_v.at[j]], sem, add=True)

      @pl.when(j >= W)
      def _():
        pltpu.make_async_copy(ones_v, acc_sh.at[dst_v.at[0]], sem).wait()
      return carry

    lax.fori_loop(0, k, chunk, 0)
    for _ in range(W):
      pltpu.make_async_copy(ones_v, acc_sh.at[dst_v.at[0]], sem).wait()
    plsc.subcore_barrier()
    pltpu.sync_copy(acc_sh.at[pl.ds(s * stripe, stripe)],
                    out_hbm.at[c, 0, pl.ds(s * stripe, stripe)])

  return pl.kernel(
      body,
      out_type=jax.ShapeDtypeStruct((NC, 8, np_), jnp.float32),
      mesh=mesh,
      scratch_types=[
          pltpu.VMEM((k, B), jnp.int32),
          pltpu.VMEM((B,), jnp.float32),
          pltpu.VMEM((stripe,), jnp.float32),
          pltpu.VMEM_SHARED((np_,), jnp.float32),
          pltpu.SemaphoreType.DMA,
      ],
      compiler_params=pltpu.CompilerParams(use_tc_tiling_on_sc=False),
      name="sc_degree_histogram",
  )


def _make_agg_kernel(n, e, d):
  k = e // NW // B
  np_ = _padded(n)
  stripe = np_ // NS
  zrep = stripe // B
  mesh = plsc.VectorSubcoreMesh(core_axis_name="c", subcore_axis_name="s")

  def body(p_hbm, src_hbm, dst_hbm, out_hbm, src_v, dst_v, rows_v, zbuf,
           acc_sh, gs0, gs1, gs2, gs3, gs4, gs5, gs6, gs7, gs8, gs9, gs10, gs11,
           ss0, ss1, ss2, ss3, ss4, ss5, ss6, ss7, ss8, ss9, ss10, ss11):
    c = lax.axis_index("c")
    s = lax.axis_index("s")
    wid = s * NC + c

    z16 = jnp.zeros((L,), jnp.float32)
    def zfill(r, carry):
      for cc in range(d // L):
        zbuf[r, pl.ds(cc * L, L)] = z16
      return carry
    lax.fori_loop(0, B, zfill, 0)
    for t in range(zrep):
      pltpu.sync_copy(zbuf, acc_sh.at[pl.ds(s * stripe + t * B, B)])
    pltpu.sync_copy(src_hbm.at[wid], src_v)
    pltpu.sync_copy(dst_hbm.at[wid], dst_v)

    gsems = (gs0, gs1, gs2, gs3, gs4, gs5, gs6, gs7, gs8, gs9, gs10, gs11)
    ssems = (ss0, ss1, ss2, ss3, ss4, ss5, ss6, ss7, ss8, ss9, ss10, ss11)
    for pj in range(6):
      pltpu.async_copy(p_hbm.at[src_v.at[pj]], rows_v.at[pj], gsems[pj])
    plsc.subcore_barrier()

    def outer(gi, carry):
      g = gi * 12
      for b in range(12):
        j = g + b
        b2 = (b + 6) % 12

        @pl.when(j < k)
        def _():
          pltpu.make_async_copy(p_hbm.at[src_v.at[j]], rows_v.at[b],
                                gsems[b]).wait()
          pltpu.async_copy(rows_v.at[b], acc_sh.at[dst_v.at[j]], ssems[b],
                           add=True)

          @pl.when(j + 6 < k)
          def _():
            @pl.when(j >= 6)
            def _():
              pltpu.make_async_copy(rows_v.at[b2],
                                    acc_sh.at[dst_v.at[j]], ssems[b2]).wait()

            pltpu.async_copy(p_hbm.at[src_v.at[j + 6]], rows_v.at[b2],
                             gsems[b2])
      return carry

    lax.fori_loop(0, (k + 11) // 12, outer, 0)
    for jj in range(k - 12, k):
      pltpu.make_async_copy(rows_v.at[jj % 12], acc_sh.at[dst_v.at[0]],
                            ssems[jj % 12]).wait()
    plsc.subcore_barrier()
    pltpu.sync_copy(acc_sh.at[pl.ds(s * stripe, stripe)],
                    out_hbm.at[c].at[pl.ds(s * stripe, stripe)])

  return pl.kernel(
      body,
      out_type=jax.ShapeDtypeStruct((NC, np_, d), jnp.float32),
      mesh=mesh,
      scratch_types=[
          pltpu.VMEM((k, B), jnp.int32),
          pltpu.VMEM((k, B), jnp.int32),
          pltpu.VMEM((12, B, d), jnp.float32),
          pltpu.VMEM((B, d), jnp.float32),
          pltpu.VMEM_SHARED((np_, d), jnp.float32),
      ] + [pltpu.SemaphoreType.DMA] * 24,
      compiler_params=pltpu.CompilerParams(use_tc_tiling_on_sc=False),
      name="sc_row_aggregate",
  )


def _tc_edge_split(ei32):
  e = ei32.shape[1]

  def body(er, o0, o1):
    o0[...] = er[0, :]
    o1[...] = er[1, :]

  return pl.pallas_call(
      body,
      out_shape=(jax.ShapeDtypeStruct((e,), jnp.int32),
                 jax.ShapeDtypeStruct((e,), jnp.int32)),
  )(ei32)


def _tc_dinv(degflat, np_):
  r = np_ // 128

  def body(dg_ref, o_ref):
    o_ref[...] = lax.rsqrt(dg_ref[0:r, :] + dg_ref[8 * r:9 * r, :] + 1.0)

  return pl.pallas_call(
      body,
      out_shape=jax.ShapeDtypeStruct((r, 128), jnp.float32),
  )(degflat)


def _tc_mm1(x, w1):
  def body(x_ref, w_ref, o_ref):
    o_ref[...] = jnp.dot(x_ref[...], w_ref[...],
                         preferred_element_type=jnp.float32)

  return pl.pallas_call(
      body,
      out_shape=jax.ShapeDtypeStruct((x.shape[0], w1.shape[1]), jnp.float32),
  )(x, w1)


def _tc_pscale(dinv, u):
  def body(d_ref, u_ref, p_ref):
    p_ref[...] = d_ref[...] * u_ref[...]

  n, h = u.shape
  g = 10
  bn = n // g
  return pl.pallas_call(
      body,
      grid=(g,),
      in_specs=[pl.BlockSpec((bn, 1), lambda i: (i, 0)),
                pl.BlockSpec((bn, h), lambda i: (i, 0))],
      out_specs=pl.BlockSpec((bn, h), lambda i: (i, 0)),
      out_shape=jax.ShapeDtypeStruct(u.shape, jnp.float32),
  )(dinv, u)


def _tc_layer1(a0, a1, p, dinv, b1, w2):
  def body(a0r, a1r, pr, dr, b1r, w2r, qr):
    ssum = a0r[...] + a1r[...] + pr[...]
    h = jnp.maximum(dr[...] * ssum + b1r[...], 0.0)
    v = jnp.dot(h, w2r[...], preferred_element_type=jnp.float32)
    qr[...] = dr[...] * v

  n = p.shape[0]
  h = p.shape[1]
  c = w2.shape[1]
  g = 10
  bn = n // g
  return pl.pallas_call(
      body,
      grid=(g,),
      in_specs=[pl.BlockSpec((bn, h), lambda i: (i, 0)),
                pl.BlockSpec((bn, h), lambda i: (i, 0)),
                pl.BlockSpec((bn, h), lambda i: (i, 0)),
                pl.BlockSpec((bn, 1), lambda i: (i, 0)),
                pl.BlockSpec((1, h), lambda i: (0, 0)),
                pl.BlockSpec((h, c), lambda i: (0, 0))],
      out_specs=pl.BlockSpec((bn, c), lambda i: (i, 0)),
      out_shape=jax.ShapeDtypeStruct((n, c), jnp.float32),
  )(a0, a1, p, dinv, b1, w2)


def _tc_final(c0, c1, q, dinv, b2):
  def body(c0r, c1r, qr, dr, b2r, outr):
    ssum = c0r[...] + c1r[...] + qr[...]
    z = (dr[...] * ssum + b2r[...]) * 5.0
    m = jnp.max(z, axis=1, keepdims=True)
    ez = jnp.exp(z - m)
    lse = jnp.log(jnp.sum(ez, axis=1, keepdims=True)) + m
    outr[...] = (z - lse).T

  n, c = q.shape
  return pl.pallas_call(
      body,
      out_shape=jax.ShapeDtypeStruct((c, n), jnp.float32),
  )(c0, c1, q, dinv, b2)


def kernel(x, edge_index, W1, b1, W2, b2):
  n, _ = x.shape
  e = edge_index.shape[1]
  h = W1.shape[1]
  c = W2.shape[1]
  k = e // NW // B
  np_ = _padded(n)

  ei = edge_index.astype(jnp.int32)
  src_f, dst_f = _tc_edge_split(ei)
  src3 = src_f.reshape(NW, k, B)
  dst3 = dst_f.reshape(NW, k, B)

  deg_part = _make_deg_kernel(n, e)(dst3)
  u = _tc_mm1(x, W1)
  degflat = deg_part.reshape(NC * 8 * np_ // 128, 128)
  dinv80 = _tc_dinv(degflat, np_)
  dinv = dinv80.reshape(np_, 1)[:n]
  p = _tc_pscale(dinv, u)
  agg1 = _make_agg_kernel(n, e, h)(p, src3, dst3)
  q = _tc_layer1(agg1[0, :n], agg1[1, :n], p, dinv,
                 b1.reshape(1, h), W2)
  agg2 = _make_agg_kernel(n, e, c)(q, src3, dst3)
  return _tc_final(agg2[0, :n], agg2[1, :n], q, dinv, b2.reshape(1, c)).T


# speedup vs baseline: 1.0313x; 1.0313x over previous
import jax
import jax.numpy as jnp
from jax import lax
from jax.experimental import pallas as pl
from jax.experimental.pallas import tpu as pltpu
from jax.experimental.pallas import tpu_sc as plsc

NC = 2
NS = 16
NW = NC * NS
B = 80
L = 16


def _padded(n):
  q = NS * 128
  return (n + q - 1) // q * q


def _make_deg_kernel(n, e):
  k = e // NW // B
  np_ = _padded(n)
  stripe = np_ // NS
  W = 8
  mesh = plsc.VectorSubcoreMesh(core_axis_name="c", subcore_axis_name="s")

  def body(dst_hbm, out_hbm, dst_v, ones_v, zbuf, acc_sh, sem):
    c = lax.axis_index("c")
    s = lax.axis_index("s")
    wid = s * NC + c

    z16 = jnp.zeros((L,), jnp.float32)
    def zfill(i, carry):
      zbuf[pl.ds(i * L, L)] = z16
      return carry
    lax.fori_loop(0, stripe // L, zfill, 0)
    for i in range(B // L):
      ones_v[pl.ds(i * L, L)] = jnp.ones((L,), jnp.float32)
    pltpu.sync_copy(zbuf, acc_sh.at[pl.ds(s * stripe, stripe)])
    pltpu.sync_copy(dst_hbm.at[wid], dst_v)
    plsc.subcore_barrier()

    def chunk(j, carry):
      pltpu.async_copy(ones_v, acc_sh.at[dst_v.at[j]], sem, add=True)

      @pl.when(j >= W)
      def _():
        pltpu.make_async_copy(ones_v, acc_sh.at[dst_v.at[0]], sem).wait()
      return carry

    lax.fori_loop(0, k, chunk, 0)
    for _ in range(W):
      pltpu.make_async_copy(ones_v, acc_sh.at[dst_v.at[0]], sem).wait()
    plsc.subcore_barrier()
    pltpu.sync_copy(acc_sh.at[pl.ds(s * stripe, stripe)],
                    out_hbm.at[c, 0, pl.ds(s * stripe, stripe)])

  return pl.kernel(
      body,
      out_type=jax.ShapeDtypeStruct((NC, 8, np_), jnp.float32),
      mesh=mesh,
      scratch_types=[
          pltpu.VMEM((k, B), jnp.int32),
          pltpu.VMEM((B,), jnp.float32),
          pltpu.VMEM((stripe,), jnp.float32),
          pltpu.VMEM_SHARED((np_,), jnp.float32),
          pltpu.SemaphoreType.DMA,
      ],
      compiler_params=pltpu.CompilerParams(use_tc_tiling_on_sc=False),
      name="sc_degree_histogram",
  )


def _make_agg_kernel(n, e, d):
  k = e // NW // B
  np_ = _padded(n)
  stripe = np_ // NS
  zrep = stripe // B
  mesh = plsc.VectorSubcoreMesh(core_axis_name="c", subcore_axis_name="s")

  def body(p_hbm, src_hbm, dst_hbm, out_hbm, src_v, dst_v, rows_v, zbuf,
           acc_sh, gs0, gs1, gs2, gs3, gs4, gs5, gs6, gs7, gs8, gs9, gs10, gs11,
           ss0, ss1, ss2, ss3, ss4, ss5, ss6, ss7, ss8, ss9, ss10, ss11):
    c = lax.axis_index("c")
    s = lax.axis_index("s")
    wid = s * NC + c

    z16 = jnp.zeros((L,), jnp.float32)
    def zfill(r, carry):
      for cc in range(d // L):
        zbuf[r, pl.ds(cc * L, L)] = z16
      return carry
    lax.fori_loop(0, B, zfill, 0)
    for t in range(zrep):
      pltpu.sync_copy(zbuf, acc_sh.at[pl.ds(s * stripe + t * B, B)])
    pltpu.sync_copy(src_hbm.at[wid], src_v)
    pltpu.sync_copy(dst_hbm.at[wid], dst_v)

    gsems = (gs0, gs1, gs2, gs3, gs4, gs5, gs6, gs7, gs8, gs9, gs10, gs11)
    ssems = (ss0, ss1, ss2, ss3, ss4, ss5, ss6, ss7, ss8, ss9, ss10, ss11)
    for pj in range(6):
      pltpu.async_copy(p_hbm.at[src_v.at[pj]], rows_v.at[pj], gsems[pj])
    plsc.subcore_barrier()

    def outer(gi, carry):
      g = gi * 12
      for b in range(12):
        j = g + b
        b2 = (b + 6) % 12

        @pl.when(j < k)
        def _():
          pltpu.make_async_copy(p_hbm.at[src_v.at[j]], rows_v.at[b],
                                gsems[b]).wait()
          pltpu.async_copy(rows_v.at[b], acc_sh.at[dst_v.at[j]], ssems[b],
                           add=True)

          @pl.when(j + 6 < k)
          def _():
            @pl.when(j >= 6)
            def _():
              pltpu.make_async_copy(rows_v.at[b2],
                                    acc_sh.at[dst_v.at[j]], ssems[b2]).wait()

            pltpu.async_copy(p_hbm.at[src_v.at[j + 6]], rows_v.at[b2],
                             gsems[b2])
      return carry

    lax.fori_loop(0, (k + 11) // 12, outer, 0)
    for jj in range(k - 12, k):
      pltpu.make_async_copy(rows_v.at[jj % 12], acc_sh.at[dst_v.at[0]],
                            ssems[jj % 12]).wait()
    plsc.subcore_barrier()
    pltpu.sync_copy(acc_sh.at[pl.ds(s * stripe, stripe)],
                    out_hbm.at[c].at[pl.ds(s * stripe, stripe)])

  return pl.kernel(
      body,
      out_type=jax.ShapeDtypeStruct((NC, np_, d), jnp.float32),
      mesh=mesh,
      scratch_types=[
          pltpu.VMEM((k, B), jnp.int32),
          pltpu.VMEM((k, B), jnp.int32),
          pltpu.VMEM((12, B, d), jnp.float32),
          pltpu.VMEM((B, d), jnp.float32),
          pltpu.VMEM_SHARED((np_, d), jnp.float32),
      ] + [pltpu.SemaphoreType.DMA] * 24,
      compiler_params=pltpu.CompilerParams(use_tc_tiling_on_sc=False),
      name="sc_row_aggregate",
  )


def _tc_edge_split(ei32):
  e = ei32.shape[1]

  def body(er, o0, o1):
    o0[...] = er[0, :]
    o1[...] = er[1, :]

  return pl.pallas_call(
      body,
      out_shape=(jax.ShapeDtypeStruct((e,), jnp.int32),
                 jax.ShapeDtypeStruct((e,), jnp.int32)),
  )(ei32)


def _tc_dinv(degflat, np_):
  r = np_ // 128

  def body(dg_ref, o_ref):
    o_ref[...] = lax.rsqrt(dg_ref[0:r, :] + dg_ref[8 * r:9 * r, :] + 1.0)

  return pl.pallas_call(
      body,
      out_shape=jax.ShapeDtypeStruct((r, 128), jnp.float32),
  )(degflat)


def _tc_mm1(x, w1):
  def body(x_ref, w_ref, o_ref):
    o_ref[...] = jnp.dot(x_ref[...], w_ref[...],
                         preferred_element_type=jnp.float32)

  return pl.pallas_call(
      body,
      out_shape=jax.ShapeDtypeStruct((x.shape[0], w1.shape[1]), jnp.float32),
  )(x, w1)


def _tc_pscale(dinv, u):
  def body(d_ref, u_ref, p_ref):
    p_ref[...] = d_ref[...] * u_ref[...]

  return pl.pallas_call(
      body,
      out_shape=jax.ShapeDtypeStruct(u.shape, jnp.float32),
  )(dinv, u)


def _tc_layer1(a0, a1, p, dinv, b1, w2):
  def body(a0r, a1r, pr, dr, b1r, w2r, qr):
    ssum = a0r[...] + a1r[...] + pr[...]
    h = jnp.maximum(dr[...] * ssum + b1r[...], 0.0)
    v = jnp.dot(h, w2r[...], preferred_element_type=jnp.float32)
    qr[...] = dr[...] * v

  n = p.shape[0]
  c = w2.shape[1]
  return pl.pallas_call(
      body,
      out_shape=jax.ShapeDtypeStruct((n, c), jnp.float32),
  )(a0, a1, p, dinv, b1, w2)


def _tc_final(c0, c1, q, dinv, b2):
  def body(c0r, c1r, qr, dr, b2r, outr):
    ssum = c0r[...] + c1r[...] + qr[...]
    z = (dr[...] * ssum + b2r[...]) * 5.0
    m = jnp.max(z, axis=1, keepdims=True)
    ez = jnp.exp(z - m)
    lse = jnp.log(jnp.sum(ez, axis=1, keepdims=True)) + m
    outr[...] = (z - lse).T

  return pl.pallas_call(
      body,
      out_shape=jax.ShapeDtypeStruct((q.shape[1], q.shape[0]), jnp.float32),
  )(c0, c1, q, dinv, b2)


def kernel(x, edge_index, W1, b1, W2, b2):
  n, _ = x.shape
  e = edge_index.shape[1]
  h = W1.shape[1]
  c = W2.shape[1]
  k = e // NW // B
  np_ = _padded(n)

  ei = edge_index.astype(jnp.int32)
  src_f, dst_f = _tc_edge_split(ei)
  src3 = src_f.reshape(NW, k, B)
  dst3 = dst_f.reshape(NW, k, B)

  deg_part = _make_deg_kernel(n, e)(dst3)
  u = _tc_mm1(x, W1)
  degflat = deg_part.reshape(NC * 8 * np_ // 128, 128)
  dinv80 = _tc_dinv(degflat, np_)
  dinv = dinv80.reshape(np_, 1)[:n]
  p = _tc_pscale(dinv, u)
  agg1 = _make_agg_kernel(n, e, h)(p, src3, dst3)
  q = _tc_layer1(agg1[0, :n], agg1[1, :n], p, dinv,
                 b1.reshape(1, h), W2)
  agg2 = _make_agg_kernel(n, e, c)(q, src3, dst3)
  return _tc_final(agg2[0, :n], agg2[1, :n], q, dinv, b2.reshape(1, c)).T
